# Initial kernel scaffold; baseline (speedup 1.0000x reference)
import jax
import jax.numpy as jnp
from jax.experimental import pallas as pl


def kernel(x, table):
    raise NotImplementedError("write your pallas kernel here")


# speedup vs baseline: 1.2458x; 1.2458x over previous
import functools
import math

import numpy as np
import jax
import jax.numpy as jnp
from jax import lax
from jax.experimental import pallas as pl
from jax.experimental.pallas import tpu as pltpu
from jax.experimental.pallas import tpu_sc as plsc

VOCAB = 32000
D = 2048
BATCH = 4
SEQ = 2048
N = BATCH * SEQ
SCALE = math.sqrt(float(D))

NC = 2
NS = 16
NW = NC * NS
BPW = N // NW
CH = 16
NCH = BPW // CH
GRP = D // 16


def _sinusoidal_pe_np(seq_len, d_model):
    pos = np.arange(seq_len, dtype=np.float64)[:, None]
    i = np.arange(0, d_model, 2, dtype=np.float64)[None, :]
    angle = pos / np.power(10000.0, i / d_model)
    pe = np.zeros((seq_len, d_model), dtype=np.float32)
    pe[:, 0::2] = np.sin(angle)
    pe[:, 1::2] = np.cos(angle)
    return pe


_PE = _sinusoidal_pe_np(SEQ, D)


def _sc_body(table_hbm, idx_hbm, pe_hbm, out_hbm, idx_v, rows_v, pe_v, gsem, psem):
    c = lax.axis_index("c")
    s = lax.axis_index("s")
    wid = s * NC + c
    base = wid * BPW
    pos0 = (wid % (SEQ // BPW)) * BPW

    pltpu.sync_copy(idx_hbm.at[wid], idx_v)

    def chunk(j, carry):
        g = pltpu.async_copy(table_hbm.at[idx_v.at[j]], rows_v, gsem)
        p = pltpu.async_copy(pe_hbm.at[pl.ds(pos0 + j * CH, CH)], pe_v, psem)
        g.wait()
        p.wait()

        def row(r, carry2):
            for grp in range(GRP):
                sl = pl.ds(grp * 16, 16)
                rows_v[r, sl] = rows_v[r, sl] * SCALE + pe_v[r, sl]
            return carry2

        lax.fori_loop(0, CH, row, 0)
        pltpu.sync_copy(rows_v, out_hbm.at[pl.ds(base + j * CH, CH)])
        return carry

    lax.fori_loop(0, NCH, chunk, 0)


@jax.jit
def _embed(x, table):
    idx = x.reshape(N).astype(jnp.int32).reshape(NW, NCH, CH)
    pe = jnp.asarray(_PE)
    mesh = plsc.VectorSubcoreMesh(core_axis_name="c", subcore_axis_name="s")
    out = pl.kernel(
        _sc_body,
        out_type=jax.ShapeDtypeStruct((N, D), jnp.float32),
        mesh=mesh,
        scratch_types=[
            pltpu.VMEM((NCH, CH), jnp.int32),
            pltpu.VMEM((CH, D), jnp.float32),
            pltpu.VMEM((CH, D), jnp.float32),
            pltpu.SemaphoreType.DMA,
            pltpu.SemaphoreType.DMA,
        ],
    )(table, idx, pe)
    return out.reshape(BATCH, SEQ, D)


def kernel(x, table):
    return _embed(x, table)
